# baseline (speedup 1.0000x reference)
import functools

import jax
import jax.numpy as jnp
from jax import lax
from jax.experimental import pallas as pl
from jax.experimental.pallas import tpu as pltpu
from jax.experimental.pallas import tpu_sc as plsc

NUM_CORES = 2
NUM_SUBCORES = 16
NUM_WORKERS = NUM_CORES * NUM_SUBCORES
CHUNK = 128
NSLOT = 5
GAHEAD = 2
ROWS_PER_ITER = 16


def kernel(indices, table):
    b_, s_ = indices.shape
    v, d = table.shape
    n = b_ * s_
    scale = float(d) ** 0.5

    rows_per_worker = n // NUM_WORKERS
    n_chunks = rows_per_worker // CHUNK

    idx_flat = indices.reshape(NUM_WORKERS, n_chunks, CHUNK).astype(jnp.int32)

    mesh = plsc.VectorSubcoreMesh(core_axis_name="c", subcore_axis_name="s")

    @functools.partial(
        pl.kernel,
        mesh=mesh,
        out_type=jax.ShapeDtypeStruct((n, d), jnp.float32),
        scratch_types=[
            pltpu.VMEM((n_chunks, CHUNK), jnp.int32),
        ] + [pltpu.VMEM((CHUNK, d), jnp.float32) for _ in range(NSLOT)]
          + [pltpu.SemaphoreType.DMA for _ in range(2 * NSLOT)],
    )
    def emb_kernel(idx_hbm, table_hbm, out_hbm, idx_v, *rest):
        bufs = list(rest[:NSLOT])
        gsem = list(rest[NSLOT:2 * NSLOT])
        ssem = list(rest[2 * NSLOT:3 * NSLOT])

        wid = lax.axis_index("s") * NUM_CORES + lax.axis_index("c")
        base = wid * rows_per_worker
        pltpu.sync_copy(idx_hbm.at[wid], idx_v)

        def issue_gather(j, slot):
            pltpu.async_copy(table_hbm.at[idx_v.at[j]], bufs[slot], gsem[slot])

        def wait_gather(slot):
            pltpu.make_async_copy(table_hbm.at[pl.ds(0, CHUNK)], bufs[slot],
                                  gsem[slot]).wait()

        def issue_scatter(j, slot):
            pltpu.async_copy(bufs[slot],
                             out_hbm.at[pl.ds(base + j * CHUNK, CHUNK)],
                             ssem[slot])

        def wait_scatter(slot):
            pltpu.make_async_copy(table_hbm.at[pl.ds(0, CHUNK)], bufs[slot],
                                  ssem[slot]).wait()

        def scale_rows(slot, lo, nrows):
            buf = bufs[slot]

            def body(i, _):
                r = lo + i * ROWS_PER_ITER
                for rr in range(ROWS_PER_ITER):
                    for c in range(d // 16):
                        sl = pl.ds(c * 16, 16)
                        buf[r + rr, sl] = buf[r + rr, sl] * scale
                return ()

            lax.fori_loop(0, nrows // ROWS_PER_ITER, body, ())

        def step(j, t, first, last):
            slot = t % NSLOT
            slot_n = (t + GAHEAD) % NSLOT
            wait_gather(slot)
            if not first:
                wait_scatter(slot_n)
            if not last:
                issue_gather(j + GAHEAD, slot_n)
            scale_rows(slot, 0, CHUNK // 2)
            pltpu.async_copy(bufs[slot].at[pl.ds(0, CHUNK // 2)],
                             out_hbm.at[pl.ds(base + j * CHUNK, CHUNK // 2)],
                             ssem[slot])
            scale_rows(slot, CHUNK // 2, CHUNK // 2)
            pltpu.async_copy(bufs[slot].at[pl.ds(CHUNK // 2, CHUNK // 2)],
                             out_hbm.at[pl.ds(base + j * CHUNK + CHUNK // 2,
                                              CHUNK // 2)],
                             ssem[slot])

        for t in range(GAHEAD):
            issue_gather(t, t)
        for t in range(NSLOT):
            step(t, t, first=(t < NSLOT - GAHEAD), last=False)

        def outer(k, _):
            jj = k * NSLOT
            for t in range(NSLOT):
                step(jj + t, t, first=False, last=False)
            return ()

        lax.fori_loop(1, n_chunks // NSLOT - 1, outer, ())

        for t in range(NSLOT):
            j = n_chunks - NSLOT + t
            step(j, t, first=False, last=(t >= NSLOT - GAHEAD))
        for j in range(n_chunks - (NSLOT - GAHEAD), n_chunks):
            wait_scatter(j % NSLOT)

    out = emb_kernel(idx_flat, table)
    return out.reshape(b_, s_, d)


# speedup vs baseline: 1.1068x; 1.1068x over previous
import functools

import jax
import jax.numpy as jnp
from jax import lax
from jax.experimental import pallas as pl
from jax.experimental.pallas import tpu as pltpu
from jax.experimental.pallas import tpu_sc as plsc

NUM_CORES = 2
NUM_SUBCORES = 16
NUM_WORKERS = NUM_CORES * NUM_SUBCORES
IDX_PER_DESC = 128
DESC_PER_SLOT = 2
SLOT_ROWS = IDX_PER_DESC * DESC_PER_SLOT
NSLOT = 3
ROWS_PER_ITER = 8


def kernel(indices, table):
    b_, s_ = indices.shape
    v, d = table.shape
    n = b_ * s_
    scale = float(d) ** 0.5

    rows_per_worker = n // NUM_WORKERS
    n_desc = rows_per_worker // IDX_PER_DESC
    n_chunks = rows_per_worker // SLOT_ROWS

    idx_flat = indices.reshape(NUM_WORKERS, n_desc, IDX_PER_DESC).astype(jnp.int32)

    mesh = plsc.VectorSubcoreMesh(core_axis_name="c", subcore_axis_name="s")

    @functools.partial(
        pl.kernel,
        mesh=mesh,
        out_type=jax.ShapeDtypeStruct((n, d), jnp.float32),
        scratch_types=[
            pltpu.VMEM((n_desc, IDX_PER_DESC), jnp.int32),
        ] + [pltpu.VMEM((SLOT_ROWS, d), jnp.float32) for _ in range(NSLOT)]
          + [pltpu.SemaphoreType.DMA for _ in range(2 * NSLOT)],
    )
    def emb_kernel(idx_hbm, table_hbm, out_hbm, idx_v, *rest):
        bufs = list(rest[:NSLOT])
        gsem = list(rest[NSLOT:2 * NSLOT])
        ssem = list(rest[2 * NSLOT:3 * NSLOT])

        wid = lax.axis_index("s") * NUM_CORES + lax.axis_index("c")
        base = wid * rows_per_worker
        pltpu.sync_copy(idx_hbm.at[wid], idx_v)

        def issue_gather(c, slot):
            for p in range(DESC_PER_SLOT):
                pltpu.async_copy(
                    table_hbm.at[idx_v.at[c * DESC_PER_SLOT + p]],
                    bufs[slot].at[pl.ds(p * IDX_PER_DESC, IDX_PER_DESC)],
                    gsem[slot])

        def wait_gather(slot):
            pltpu.make_async_copy(table_hbm.at[pl.ds(0, SLOT_ROWS)],
                                  bufs[slot], gsem[slot]).wait()

        def issue_scatter(c, slot):
            pltpu.async_copy(bufs[slot],
                             out_hbm.at[pl.ds(base + c * SLOT_ROWS, SLOT_ROWS)],
                             ssem[slot])

        def wait_scatter(slot):
            pltpu.make_async_copy(table_hbm.at[pl.ds(0, SLOT_ROWS)],
                                  bufs[slot], ssem[slot]).wait()

        def scale_slot(slot):
            buf = bufs[slot]

            def body(i, _):
                r = i * ROWS_PER_ITER
                for rr in range(ROWS_PER_ITER):
                    for cc in range(d // 16):
                        sl = pl.ds(cc * 16, 16)
                        buf[r + rr, sl] = buf[r + rr, sl] * scale
                return ()

            lax.fori_loop(0, SLOT_ROWS // ROWS_PER_ITER, body, ())

        def step(c, t, first, last):
            slot = t % NSLOT
            slot_n = (t + 1) % NSLOT
            wait_gather(slot)
            if not first:
                wait_scatter(slot_n)
            if not last:
                issue_gather(c + 1, slot_n)
            scale_slot(slot)
            issue_scatter(c, slot)

        issue_gather(0, 0)
        for t in range(NSLOT):
            step(t, t, first=(t < 2), last=False)

        n_main = (n_chunks - 1 - NSLOT) // NSLOT

        def outer(k, _):
            jj = k * NSLOT
            for t in range(NSLOT):
                step(jj + t, t, first=False, last=False)
            return ()

        lax.fori_loop(1, 1 + n_main, outer, ())

        for c in range(NSLOT + n_main * NSLOT, n_chunks):
            step(c, c, first=False, last=(c == n_chunks - 1))
        for c in range(n_chunks - 2, n_chunks):
            wait_scatter(c % NSLOT)

    out = emb_kernel(idx_flat, table)
    return out.reshape(b_, s_, d)


# speedup vs baseline: 1.1104x; 1.0032x over previous
import functools

import jax
import jax.numpy as jnp
from jax import lax
from jax.experimental import pallas as pl
from jax.experimental.pallas import tpu as pltpu
from jax.experimental.pallas import tpu_sc as plsc

NUM_CORES = 2
NUM_SUBCORES = 16
NUM_WORKERS = NUM_CORES * NUM_SUBCORES
IDX_PER_DESC = 128
DESC_PER_SLOT = 2
SLOT_ROWS = IDX_PER_DESC * DESC_PER_SLOT
NSLOT = 3
ROWS_PER_ITER = 8


def kernel(indices, table):
    b_, s_ = indices.shape
    v, d = table.shape
    n = b_ * s_
    scale = float(d) ** 0.5

    rows_per_worker = n // NUM_WORKERS
    n_desc = rows_per_worker // IDX_PER_DESC
    n_chunks = rows_per_worker // SLOT_ROWS

    idx_flat = indices.reshape(NUM_WORKERS, n_desc, IDX_PER_DESC).astype(jnp.int32)

    mesh = plsc.VectorSubcoreMesh(core_axis_name="c", subcore_axis_name="s")

    @functools.partial(
        pl.kernel,
        mesh=mesh,
        out_type=jax.ShapeDtypeStruct((n, d), jnp.float32),
        scratch_types=[
            pltpu.VMEM((n_desc, IDX_PER_DESC), jnp.int32),
        ] + [pltpu.VMEM((SLOT_ROWS, d), jnp.float32) for _ in range(NSLOT)]
          + [pltpu.SemaphoreType.DMA for _ in range(2 * NSLOT)],
    )
    def emb_kernel(idx_hbm, table_hbm, out_hbm, idx_v, *rest):
        bufs = list(rest[:NSLOT])
        gsem = list(rest[NSLOT:2 * NSLOT])
        ssem = list(rest[2 * NSLOT:3 * NSLOT])

        wid = lax.axis_index("s") * NUM_CORES + lax.axis_index("c")
        base = wid * rows_per_worker
        pltpu.sync_copy(idx_hbm.at[wid], idx_v)

        def issue_gather(c, slot):
            for p in range(DESC_PER_SLOT):
                pltpu.async_copy(
                    table_hbm.at[idx_v.at[c * DESC_PER_SLOT + p]],
                    bufs[slot].at[pl.ds(p * IDX_PER_DESC, IDX_PER_DESC)],
                    gsem[slot])

        def wait_gather_half(slot, p):
            pltpu.make_async_copy(
                table_hbm.at[pl.ds(0, IDX_PER_DESC)],
                bufs[slot].at[pl.ds(p * IDX_PER_DESC, IDX_PER_DESC)],
                gsem[slot]).wait()

        def issue_scatter(c, slot):
            pltpu.async_copy(bufs[slot],
                             out_hbm.at[pl.ds(base + c * SLOT_ROWS, SLOT_ROWS)],
                             ssem[slot])

        def wait_scatter(slot):
            pltpu.make_async_copy(table_hbm.at[pl.ds(0, SLOT_ROWS)],
                                  bufs[slot], ssem[slot]).wait()

        def scale_half(slot, p):
            buf = bufs[slot]

            def body(i, _):
                r = p * IDX_PER_DESC + i * ROWS_PER_ITER
                for rr in range(ROWS_PER_ITER):
                    for cc in range(d // 16):
                        sl = pl.ds(cc * 16, 16)
                        buf[r + rr, sl] = buf[r + rr, sl] * scale
                return ()

            lax.fori_loop(0, IDX_PER_DESC // ROWS_PER_ITER, body, ())

        def step(c, t, first, last):
            slot = t % NSLOT
            slot_n = (t + 1) % NSLOT
            wait_gather_half(slot, 0)
            if not first:
                wait_scatter(slot_n)
            if not last:
                issue_gather(c + 1, slot_n)
            scale_half(slot, 0)
            wait_gather_half(slot, 1)
            scale_half(slot, 1)
            issue_scatter(c, slot)

        issue_gather(0, 0)
        for t in range(NSLOT):
            step(t, t, first=(t < 2), last=False)

        n_main = (n_chunks - 1 - NSLOT) // NSLOT

        def outer(k, _):
            jj = k * NSLOT
            for t in range(NSLOT):
                step(jj + t, t, first=False, last=False)
            return ()

        lax.fori_loop(1, 1 + n_main, outer, ())

        for c in range(NSLOT + n_main * NSLOT, n_chunks):
            step(c, c, first=False, last=(c == n_chunks - 1))
        for c in range(n_chunks - 2, n_chunks):
            wait_scatter(c % NSLOT)

    out = emb_kernel(idx_flat, table)
    return out.reshape(b_, s_, d)


# speedup vs baseline: 1.1209x; 1.0095x over previous
import functools

import jax
import jax.numpy as jnp
from jax import lax
from jax.experimental import pallas as pl
from jax.experimental.pallas import tpu as pltpu
from jax.experimental.pallas import tpu_sc as plsc

NUM_CORES = 2
NUM_SUBCORES = 16
NUM_WORKERS = NUM_CORES * NUM_SUBCORES
IDX_PER_DESC = 128
DESC_PER_SLOT = 2
SLOT_ROWS = IDX_PER_DESC * DESC_PER_SLOT
NSLOT = 3
ROWS_PER_ITER = 8


def kernel(indices, table):
    b_, s_ = indices.shape
    v, d = table.shape
    n = b_ * s_
    scale = float(d) ** 0.5

    rows_per_worker = n // NUM_WORKERS
    n_desc = rows_per_worker // IDX_PER_DESC
    n_chunks = rows_per_worker // SLOT_ROWS

    idx_flat = indices.reshape(NUM_WORKERS, n_desc, IDX_PER_DESC).astype(jnp.int32)

    mesh = plsc.VectorSubcoreMesh(core_axis_name="c", subcore_axis_name="s")

    @functools.partial(
        pl.kernel,
        mesh=mesh,
        out_type=jax.ShapeDtypeStruct((n, d), jnp.float32),
        scratch_types=[
            pltpu.VMEM((n_desc, IDX_PER_DESC), jnp.int32),
        ] + [pltpu.VMEM((SLOT_ROWS, d), jnp.float32) for _ in range(NSLOT)]
          + [pltpu.SemaphoreType.DMA for _ in range(2 * NSLOT)],
    )
    def emb_kernel(idx_hbm, table_hbm, out_hbm, idx_v, *rest):
        bufs = list(rest[:NSLOT])
        gsem = list(rest[NSLOT:2 * NSLOT])
        ssem = list(rest[2 * NSLOT:3 * NSLOT])

        wid = lax.axis_index("s") * NUM_CORES + lax.axis_index("c")
        base = wid * rows_per_worker
        pltpu.sync_copy(idx_hbm.at[wid, pl.ds(0, 8)], idx_v.at[pl.ds(0, 8)])

        def issue_gather(c, slot):
            for p in range(DESC_PER_SLOT):
                pltpu.async_copy(
                    table_hbm.at[idx_v.at[c * DESC_PER_SLOT + p]],
                    bufs[slot].at[pl.ds(p * IDX_PER_DESC, IDX_PER_DESC)],
                    gsem[slot])

        def wait_gather_half(slot, p):
            pltpu.make_async_copy(
                table_hbm.at[pl.ds(0, IDX_PER_DESC)],
                bufs[slot].at[pl.ds(p * IDX_PER_DESC, IDX_PER_DESC)],
                gsem[slot]).wait()

        def issue_scatter(c, slot):
            pltpu.async_copy(bufs[slot],
                             out_hbm.at[pl.ds(base + c * SLOT_ROWS, SLOT_ROWS)],
                             ssem[slot])

        def wait_scatter(slot):
            pltpu.make_async_copy(table_hbm.at[pl.ds(0, SLOT_ROWS)],
                                  bufs[slot], ssem[slot]).wait()

        def scale_half(slot, p):
            buf = bufs[slot]

            def body(i, _):
                r = p * IDX_PER_DESC + i * ROWS_PER_ITER
                for rr in range(ROWS_PER_ITER):
                    for cc in range(d // 16):
                        sl = pl.ds(cc * 16, 16)
                        buf[r + rr, sl] = buf[r + rr, sl] * scale
                return ()

            lax.fori_loop(0, IDX_PER_DESC // ROWS_PER_ITER, body, ())

        def step(c, t, first, last):
            slot = t % NSLOT
            slot_n = (t + 1) % NSLOT
            wait_gather_half(slot, 0)
            if not first:
                wait_scatter(slot_n)
            if not last:
                issue_gather(c + 1, slot_n)
            scale_half(slot, 0)
            wait_gather_half(slot, 1)
            scale_half(slot, 1)
            issue_scatter(c, slot)

        issue_gather(0, 0)
        pltpu.sync_copy(idx_hbm.at[wid, pl.ds(8, n_desc - 8)],
                        idx_v.at[pl.ds(8, n_desc - 8)])
        for t in range(NSLOT):
            step(t, t, first=(t < 2), last=False)

        n_main = (n_chunks - 1 - NSLOT) // NSLOT

        def outer(k, _):
            jj = k * NSLOT
            for t in range(NSLOT):
                step(jj + t, t, first=False, last=False)
            return ()

        lax.fori_loop(1, 1 + n_main, outer, ())

        for c in range(NSLOT + n_main * NSLOT, n_chunks):
            step(c, c, first=False, last=(c == n_chunks - 1))
        for c in range(n_chunks - 2, n_chunks):
            wait_scatter(c % NSLOT)

    out = emb_kernel(idx_flat, table)
    return out.reshape(b_, s_, d)
